# baseline (speedup 1.0000x reference)
import functools
import math

import jax
import jax.numpy as jnp
import numpy as np
from jax.experimental import pallas as pl
from jax.experimental.pallas import tpu as pltpu

_NBUF = 4
_NSPLIT = 5


def _encoder_pipe(emb_ref, times_ref, lens_ref, freq_ref, out_ref,
                  in_buf, out_buf, in_sems, out_sems,
                  *, nb, n, e, scale, smax):
    for j in range(_NBUF):
        pltpu.make_async_copy(
            emb_ref.at[pl.ds(j * (nb // _NBUF), nb // _NBUF)],
            out_ref.at[pl.ds(j * (nb // _NBUF), nb // _NBUF)],
            in_sems.at[j, 0]).start()
    for j in range(_NBUF):
        pltpu.make_async_copy(
            emb_ref.at[pl.ds(j * (nb // _NBUF), nb // _NBUF)],
            out_ref.at[pl.ds(j * (nb // _NBUF), nb // _NBUF)],
            in_sems.at[j, 0]).wait()


def kernel(embeddings, times, sequence_lengths, sinusoidal_table):
    B, T, N, E = embeddings.shape
    S = sinusoidal_table.shape[0]
    scale = math.sqrt(E)

    div = np.exp(np.arange(0, E, 2, dtype=np.float32) *
                 (-math.log(10000.0) / E))
    freq = jnp.asarray(np.repeat(div, 2).reshape(1, E))

    out = pl.pallas_call(
        functools.partial(_encoder_pipe, nb=B, n=N, e=E, scale=scale,
                          smax=S - 1),
        in_specs=[
            pl.BlockSpec(memory_space=pl.ANY),
            pl.BlockSpec(memory_space=pltpu.VMEM),
            pl.BlockSpec(memory_space=pltpu.SMEM),
            pl.BlockSpec(memory_space=pltpu.VMEM),
        ],
        out_specs=pl.BlockSpec(memory_space=pl.ANY),
        out_shape=jax.ShapeDtypeStruct((B, T, N * E), jnp.float32),
        scratch_shapes=[
            pltpu.VMEM((_NBUF, T, N * E), jnp.float32),
            pltpu.VMEM((_NBUF, T, N * E), jnp.float32),
            pltpu.SemaphoreType.DMA((_NBUF, _NSPLIT)),
            pltpu.SemaphoreType.DMA((_NBUF, _NSPLIT)),
        ],
    )(embeddings.reshape(B, T, N * E), times.reshape(B, T, 1),
      sequence_lengths.astype(jnp.int32), freq)
    return out.reshape(B, T, N, E)


# speedup vs baseline: 12.6420x; 12.6420x over previous
import functools
import math

import jax
import jax.numpy as jnp
import numpy as np
from jax.experimental import pallas as pl
from jax.experimental.pallas import tpu as pltpu


def _encoder_block(lens_sm, emb_ref, times_ref, freq_ref, out_ref,
                   *, bb, scale, smax):
    b0 = pl.program_id(0) * bb
    T = emb_ref.shape[1]

    for kb in range(bb):
        b = b0 + kb
        tv = times_ref[b]
        idxf = jnp.clip(jnp.round(tv * 10.0), 0.0, float(smax))
        angle = idxf * freq_ref[...]
        lane = jax.lax.broadcasted_iota(jnp.int32, angle.shape, 1)
        row = jnp.where(lane % 2 == 0, jnp.sin(angle), jnp.cos(angle))

        seqlen = lens_sm[b]
        tvec = jax.lax.broadcasted_iota(jnp.int32, (T, 1), 0)
        valid = (tvec < seqlen).astype(jnp.float32)
        sin_embed = row * valid

        out_ref[kb] = emb_ref[kb] * scale + sin_embed[:, None, :]


def kernel(embeddings, times, sequence_lengths, sinusoidal_table):
    B, T, N, E = embeddings.shape
    S = sinusoidal_table.shape[0]
    scale = math.sqrt(E)
    bb = 2

    div = np.exp(np.arange(0, E, 2, dtype=np.float32) *
                 (-math.log(10000.0) / E))
    freq = jnp.asarray(np.repeat(div, 2).reshape(1, E))

    grid_spec = pltpu.PrefetchScalarGridSpec(
        num_scalar_prefetch=1,
        grid=(B // bb,),
        in_specs=[
            pl.BlockSpec((bb, T, N, E), lambda b, *_: (b, 0, 0, 0)),
            pl.BlockSpec((B, T, 1), lambda b, *_: (0, 0, 0)),
            pl.BlockSpec((1, E), lambda b, *_: (0, 0)),
        ],
        out_specs=pl.BlockSpec((bb, T, N, E), lambda b, *_: (b, 0, 0, 0)),
    )

    return pl.pallas_call(
        functools.partial(_encoder_block, bb=bb, scale=scale, smax=S - 1),
        grid_spec=grid_spec,
        out_shape=jax.ShapeDtypeStruct((B, T, N, E), jnp.float32),
    )(sequence_lengths.astype(jnp.int32), embeddings,
      times.reshape(B, T, 1), freq)
